# Initial kernel scaffold; baseline (speedup 1.0000x reference)
import jax
import jax.numpy as jnp
from jax.experimental import pallas as pl


def kernel(x, edge_index, edge_attr, params):
    raise NotImplementedError("write your pallas kernel here")


# speedup vs baseline: 5.6146x; 5.6146x over previous
import functools

import jax
import jax.numpy as jnp
from jax import lax
from jax.experimental import pallas as pl
from jax.experimental.pallas import tpu as pltpu
from jax.experimental.pallas import tpu_sc as plsc

N = 10000
E = 320000
D_FEAT = 128
C = 64
FINAL = 3

NC, NS, LANES = 2, 16, 16
NW = NC * NS
CH = 80
RP = NW * CH
NPAD = N + 240
NSL = NPAD // NS

BN = 1000
NB = N // BN
RB = 320
PB = RP // RB

_SQRT_HALF = 0.7071067811865476
_DOT = dict(preferred_element_type=jnp.float32, precision=lax.Precision.HIGHEST)


def _gelu(z):
    return 0.5 * z * (1.0 + lax.erf(z * _SQRT_HALF))


def _prep_body(src_ref, dst_ref, attr_ref, srcp_ref, dstp_ref):
    src = src_ref[...]
    dst = dst_ref[...]
    attr = attr_ref[...]
    base = pl.program_id(0) * (RB * 128)
    epos = base + (lax.broadcasted_iota(jnp.int32, (RB, 128), 0) * 128
                   + lax.broadcasted_iota(jnp.int32, (RB, 128), 1))
    is_pad = epos >= E
    dummy = N + (epos & 127)
    s0 = jnp.where(is_pad, epos - E, src)
    srcp_ref[0] = s0
    srcp_ref[1] = s0 + N
    srcp_ref[2] = s0 + 2 * N
    dstp_ref[0] = jnp.where(jnp.logical_and(~is_pad, attr <= 0.0), dst, dummy)
    dstp_ref[1] = jnp.where(jnp.logical_and(~is_pad, attr >= 0.0), dst, dummy)
    dstp_ref[2] = jnp.where(is_pad, dummy, dst)


def _prep(srcr, dstr, attrr):
    eb = pl.BlockSpec((RB, 128), lambda i: (i, 0))
    ob = pl.BlockSpec((3, RB, 128), lambda i: (0, i, 0))
    return pl.pallas_call(
        _prep_body,
        grid=(PB,),
        in_specs=[eb, eb, eb],
        out_specs=(ob, ob),
        out_shape=(jax.ShapeDtypeStruct((3, RP, 128), jnp.int32),
                   jax.ShapeDtypeStruct((3, RP, 128), jnp.int32)),
    )(srcr, dstr, attrr)


def _sums(t):
    return jnp.concatenate([jnp.sum(t, 0, keepdims=True),
                            jnp.sum(t * t, 0, keepdims=True)], axis=0)


def _dense0a_body(x_ref, w_ref, b_ref, t_ref, ps_ref):
    t = jnp.dot(x_ref[...], w_ref[...], **_DOT) + b_ref[...]
    t_ref[...] = t
    ps_ref[0] = _sums(t)


def _norm_gelu(t, ps, g, gb, a, eps=1e-5):
    s1 = jnp.sum(ps[:, 0, :], axis=0, keepdims=True) * (1.0 / N)
    s2 = jnp.sum(ps[:, 1, :], axis=0, keepdims=True) * (1.0 / N)
    var = s2 - a * (2.0 - a) * s1 * s1
    o = t - a * s1
    return _gelu(o * lax.rsqrt(var + eps) * g + gb)


def _dense0b_body(t_ref, ps_ref, g_ref, gb_ref, ga_ref, h_ref):
    h_ref[...] = _norm_gelu(t_ref[...], ps_ref[...], g_ref[...], gb_ref[...],
                            ga_ref[...])


def _dense0(x, w, b, g, gb, ga):
    nb64 = pl.BlockSpec((BN, C), lambda i: (i, 0))
    v64 = pl.BlockSpec((1, C), lambda i: (0, 0))
    t0, ps0 = pl.pallas_call(
        _dense0a_body,
        grid=(NB,),
        in_specs=[pl.BlockSpec((BN, D_FEAT), lambda i: (i, 0)),
                  pl.BlockSpec((D_FEAT, C), lambda i: (0, 0)), v64],
        out_specs=(nb64, pl.BlockSpec((1, 2, C), lambda i: (i, 0, 0))),
        out_shape=(jax.ShapeDtypeStruct((N, C), jnp.float32),
                   jax.ShapeDtypeStruct((NB, 2, C), jnp.float32)),
    )(x, w, b)
    return pl.pallas_call(
        _dense0b_body,
        grid=(NB,),
        in_specs=[nb64, pl.BlockSpec((NB, 2, C), lambda i: (0, 0, 0)),
                  v64, v64, v64],
        out_specs=nb64,
        out_shape=jax.ShapeDtypeStruct((N, C), jnp.float32),
    )(t0, ps0, g, gb, ga)


def _agg_block(p_ref, cnt_ref):
    cnt = jnp.maximum(cnt_ref[0, 0] + cnt_ref[1, 0], 1.0)
    return (p_ref[0, 0] + p_ref[1, 0]) / cnt


def _dense_mid_a_body(h_ref, p_ref, cnt_ref, wr_ref, wn_ref, b_ref,
                      t_ref, ps_ref):
    t = (jnp.dot(h_ref[0], wr_ref[0], **_DOT)
         + jnp.dot(_agg_block(p_ref, cnt_ref), wn_ref[0], **_DOT) + b_ref[0])
    t_ref[0] = t
    ps_ref[0, 0] = _sums(t)


def _dense_mid_b_body(t_ref, ps_ref, g_ref, gb_ref, ga_ref, z_ref):
    z_ref[0] = _norm_gelu(t_ref[0], ps_ref[0], g_ref[0], gb_ref[0], ga_ref[0])


def _dense_out_body(z_ref, p_ref, cnt_ref, wr_ref, wn_ref, b_ref, o_ref):
    o_ref[0] = (jnp.dot(z_ref[0], wr_ref[0], **_DOT)
                + jnp.dot(_agg_block(p_ref, cnt_ref), wn_ref[0], **_DOT)
                + b_ref[0])


def _bspec(per_branch, shape):
    if per_branch:
        return pl.BlockSpec((1,) + shape, lambda m, i: (m,) + (0,) * len(shape))
    return pl.BlockSpec((1,) + shape, lambda m, i: (0,) * (len(shape) + 1))


_P_SPEC = pl.BlockSpec((NC, 1, BN, C), lambda m, i: (0, m, i, 0))
_CNT_SPEC = pl.BlockSpec((NC, 1, BN, 1), lambda m, i: (0, m, i, 0))


def _in_spec(per_branch_in):
    if per_branch_in:
        return pl.BlockSpec((1, BN, C), lambda m, i: (m, i, 0))
    return pl.BlockSpec((1, BN, C), lambda m, i: (0, i, 0))


def _dense_mid(h, p, cnt, wr, wn, b, g, gb, ga, per_branch_w, per_branch_in):
    zspec = pl.BlockSpec((1, BN, C), lambda m, i: (m, i, 0))
    t, ps = pl.pallas_call(
        _dense_mid_a_body,
        grid=(3, NB),
        in_specs=[_in_spec(per_branch_in), _P_SPEC, _CNT_SPEC,
                  _bspec(per_branch_w, (C, C)), _bspec(per_branch_w, (C, C)),
                  _bspec(per_branch_w, (1, C))],
        out_specs=(zspec, pl.BlockSpec((1, 1, 2, C), lambda m, i: (m, i, 0, 0))),
        out_shape=(jax.ShapeDtypeStruct((3, N, C), jnp.float32),
                   jax.ShapeDtypeStruct((3, NB, 2, C), jnp.float32)),
    )(h, p, cnt, wr, wn, b)
    return pl.pallas_call(
        _dense_mid_b_body,
        grid=(3, NB),
        in_specs=[zspec, pl.BlockSpec((1, NB, 2, C), lambda m, i: (m, 0, 0, 0)),
                  _bspec(per_branch_w, (1, C)), _bspec(per_branch_w, (1, C)),
                  _bspec(per_branch_w, (1, C))],
        out_specs=zspec,
        out_shape=jax.ShapeDtypeStruct((3, N, C), jnp.float32),
    )(t, ps, g, gb, ga)


def _dense_out(z, p, cnt, wr, wn, b):
    return pl.pallas_call(
        _dense_out_body,
        grid=(3, NB),
        in_specs=[pl.BlockSpec((1, BN, C), lambda m, i: (m, i, 0)),
                  _P_SPEC, _CNT_SPEC,
                  _bspec(True, (C, FINAL)), _bspec(True, (C, FINAL)),
                  _bspec(True, (1, FINAL))],
        out_specs=pl.BlockSpec((1, BN, FINAL), lambda m, i: (m, i, 0)),
        out_shape=jax.ShapeDtypeStruct((3, N, FINAL), jnp.float32),
    )(z, p, cnt, wr, wn, b)


@functools.cache
def _mesh():
    return plsc.VectorSubcoreMesh(core_axis_name="c", subcore_axis_name="s",
                                  num_cores=NC, num_subcores=NS)


def _sc_counts_body(dstp_hbm, ones_hbm, zeros_hbm, out_hbm,
                    dst_v, ones_v, acc0, acc1, acc2, sem):
    cid = lax.axis_index("c")
    sid = lax.axis_index("s")
    w = sid * NC + cid
    accs = (acc0, acc1, acc2)
    for acc in accs:
        pltpu.sync_copy(zeros_hbm.at[pl.ds(sid * NSL, NSL)],
                        acc.at[pl.ds(sid * NSL, NSL)])
    pltpu.sync_copy(ones_hbm, ones_v)
    for m in range(3):
        pltpu.sync_copy(dstp_hbm.at[m, pl.ds(w * CH, CH)],
                        dst_v.at[pl.ds(m * CH, CH)])
    plsc.subcore_barrier()
    for m, acc in enumerate(accs):
        @pl.loop(0, CH)
        def _(c):
            pltpu.sync_copy(ones_v, acc.at[dst_v.at[m * CH + c]], add=True)
    plsc.subcore_barrier()
    for m, acc in enumerate(accs):
        pltpu.sync_copy(acc.at[pl.ds(sid * NSL, NSL)],
                        out_hbm.at[cid, m, pl.ds(sid * NSL, NSL)])


def _sc_agg_body(stage1, table_hbm, srcp_hbm, dstp_hbm, zeros_hbm, out_hbm,
                 src_v, dst_v, rows_v, acc, sem):
    cid = lax.axis_index("c")
    sid = lax.axis_index("s")
    w = sid * NC + cid
    for m in range(3):
        pltpu.sync_copy(dstp_hbm.at[m, pl.ds(w * CH, CH)],
                        dst_v.at[pl.ds(m * CH, CH)])
        sm = 0 if stage1 else m
        pltpu.sync_copy(srcp_hbm.at[sm, pl.ds(w * CH, CH)],
                        src_v.at[pl.ds(m * CH, CH)])
    for m in range(3):
        pltpu.sync_copy(zeros_hbm.at[pl.ds(sid * NSL, NSL)],
                        acc.at[pl.ds(sid * NSL, NSL)])
        plsc.subcore_barrier()

        @pl.loop(0, CH)
        def _(c):
            pltpu.async_copy(table_hbm.at[src_v.at[m * CH + c]], rows_v,
                             sem).wait()
            pltpu.sync_copy(rows_v, acc.at[dst_v.at[m * CH + c]], add=True)

        plsc.subcore_barrier()
        pltpu.sync_copy(acc.at[pl.ds(sid * NSL, NSL)],
                        out_hbm.at[cid, m, pl.ds(sid * NSL, NSL)])


def _make_sc_counts():
    return pl.kernel(
        _sc_counts_body,
        out_type=jax.ShapeDtypeStruct((NC, 3, NPAD, 16), jnp.float32),
        mesh=_mesh(),
        compiler_params=pltpu.CompilerParams(use_tc_tiling_on_sc=False),
        scratch_types=[
            pltpu.VMEM((3 * CH, 128), jnp.int32),
            pltpu.VMEM((128, 16), jnp.float32),
            pltpu.VMEM_SHARED((NPAD, 16), jnp.float32),
            pltpu.VMEM_SHARED((NPAD, 16), jnp.float32),
            pltpu.VMEM_SHARED((NPAD, 16), jnp.float32),
            pltpu.SemaphoreType.DMA,
        ],
    )


def _make_sc_agg(stage1):
    return pl.kernel(
        functools.partial(_sc_agg_body, stage1),
        out_type=jax.ShapeDtypeStruct((NC, 3, NPAD, C), jnp.float32),
        mesh=_mesh(),
        compiler_params=pltpu.CompilerParams(use_tc_tiling_on_sc=False),
        scratch_types=[
            pltpu.VMEM((3 * CH, 128), jnp.int32),
            pltpu.VMEM((3 * CH, 128), jnp.int32),
            pltpu.VMEM((128, C), jnp.float32),
            pltpu.VMEM_SHARED((NPAD, C), jnp.float32),
            pltpu.SemaphoreType.DMA,
        ],
    )


def kernel(x, edge_index, edge_attr, params):
    pad = RP * 128 - E
    srcr = jnp.pad(edge_index[0], (0, pad)).reshape(RP, 128)
    dstr = jnp.pad(edge_index[1], (0, pad)).reshape(RP, 128)
    attrr = jnp.pad(edge_attr, (0, pad)).reshape(RP, 128)
    srcp, dstp = _prep(srcr, dstr, attrr)

    p011 = params["layer011"]
    gn0 = params["batch01"]
    h = _dense0(x, p011["w"], p011["b"].reshape(1, C),
                gn0["g"].reshape(1, C), gn0["b"].reshape(1, C),
                gn0["a"].reshape(1, C))

    zeros_c = jnp.zeros((NPAD, C), jnp.float32)
    zeros16 = jnp.zeros((NPAD, 16), jnp.float32)
    ones16 = jnp.ones((128, 16), jnp.float32)

    cntp = _make_sc_counts()(dstp, ones16, zeros16)
    cnt1 = cntp[:, :, :, 0].reshape(NC, 3, NPAD, 1)
    agg1 = _make_sc_agg(True)(h, srcp, dstp, zeros_c)

    def stack(names, field):
        return jnp.stack([params[n][field] for n in names])

    l1 = ("layer11", "layer12", "layer13")
    g1 = ("batch11", "batch12", "batch13")
    z1 = _dense_mid(h[None], agg1, cnt1,
                    stack(l1, "wr"), stack(l1, "wn"),
                    stack(l1, "b")[:, None, :],
                    stack(g1, "g")[:, None, :], stack(g1, "b")[:, None, :],
                    stack(g1, "a")[:, None, :],
                    per_branch_w=True, per_branch_in=False)

    agg2 = _make_sc_agg(False)(z1.reshape(3 * N, C), srcp, dstp, zeros_c)

    p21 = params["layer21"]
    g21 = params["batch21"]
    z2 = _dense_mid(z1, agg2, cnt1,
                    p21["wr"][None], p21["wn"][None],
                    p21["b"].reshape(1, 1, C),
                    g21["g"].reshape(1, 1, C), g21["b"].reshape(1, 1, C),
                    g21["a"].reshape(1, 1, C),
                    per_branch_w=False, per_branch_in=True)

    agg3 = _make_sc_agg(False)(z2.reshape(3 * N, C), srcp, dstp, zeros_c)

    l3 = ("layer31", "layer32", "layer33")
    out = _dense_out(z2, agg3, cnt1,
                     stack(l3, "wr"), stack(l3, "wn"),
                     stack(l3, "b")[:, None, :])
    return (out[0], out[1], out[2])


# speedup vs baseline: 8.1906x; 1.4588x over previous
import functools

import jax
import jax.numpy as jnp
from jax import lax
from jax.experimental import pallas as pl
from jax.experimental.pallas import tpu as pltpu
from jax.experimental.pallas import tpu_sc as plsc

N = 10000
E = 320000
D_FEAT = 128
C = 64
FINAL = 3

NC, NS, LANES = 2, 16, 16
NW = NC * NS
CH = 80
RP = NW * CH
NPAD = N + 240
NSL = NPAD // NS

BN = 1000
NB = N // BN
RB = 320
PB = RP // RB

_SQRT_HALF = 0.7071067811865476
_DOT = dict(preferred_element_type=jnp.float32, precision=lax.Precision.HIGHEST)


def _gelu(z):
    return 0.5 * z * (1.0 + lax.erf(z * _SQRT_HALF))


def _prep_body(src_ref, dst_ref, attr_ref, srcp_ref, dstp_ref):
    src = src_ref[...]
    dst = dst_ref[...]
    attr = attr_ref[...]
    base = pl.program_id(0) * (RB * 128)
    epos = base + (lax.broadcasted_iota(jnp.int32, (RB, 128), 0) * 128
                   + lax.broadcasted_iota(jnp.int32, (RB, 128), 1))
    is_pad = epos >= E
    dummy = N + (epos & 127)
    s0 = jnp.where(is_pad, epos - E, src)
    srcp_ref[0] = s0
    srcp_ref[1] = s0 + N
    srcp_ref[2] = s0 + 2 * N
    dstp_ref[0] = jnp.where(jnp.logical_and(~is_pad, attr <= 0.0), dst, dummy)
    dstp_ref[1] = jnp.where(jnp.logical_and(~is_pad, attr >= 0.0), dst, dummy)
    dstp_ref[2] = jnp.where(is_pad, dummy, dst)


def _prep(srcr, dstr, attrr):
    eb = pl.BlockSpec((RB, 128), lambda i: (i, 0))
    ob = pl.BlockSpec((3, RB, 128), lambda i: (0, i, 0))
    return pl.pallas_call(
        _prep_body,
        grid=(PB,),
        in_specs=[eb, eb, eb],
        out_specs=(ob, ob),
        out_shape=(jax.ShapeDtypeStruct((3, RP, 128), jnp.int32),
                   jax.ShapeDtypeStruct((3, RP, 128), jnp.int32)),
    )(srcr, dstr, attrr)


def _sums(t):
    return jnp.concatenate([jnp.sum(t, 0, keepdims=True),
                            jnp.sum(t * t, 0, keepdims=True)], axis=0)


def _norm_gelu(t, ps, g, gb, a, eps=1e-5):
    s1 = jnp.sum(ps[:, 0, :], axis=0, keepdims=True) * (1.0 / N)
    s2 = jnp.sum(ps[:, 1, :], axis=0, keepdims=True) * (1.0 / N)
    var = s2 - a * (2.0 - a) * s1 * s1
    o = t - a * s1
    return _gelu(o * lax.rsqrt(var + eps) * g + gb)


def _agg_block(p_ref, cnt_ref):
    cnt = jnp.maximum(cnt_ref[0, 0] + cnt_ref[1, 0], 1.0)
    return (p_ref[0, 0] + p_ref[1, 0]) / cnt


def _dense0(x, w, b, g, gb, ga):
    return pl.pallas_call(
        _dense0a_wrap,
        grid=(2, NB),
        in_specs=[pl.BlockSpec((BN, D_FEAT), lambda ph, i: (i * (1 - ph), 0)),
                  pl.BlockSpec((D_FEAT, C), lambda ph, i: (0, 0)),
                  pl.BlockSpec((1, C), lambda ph, i: (0, 0)),
                  pl.BlockSpec((1, C), lambda ph, i: (0, 0)),
                  pl.BlockSpec((1, C), lambda ph, i: (0, 0)),
                  pl.BlockSpec((1, C), lambda ph, i: (0, 0))],
        out_specs=pl.BlockSpec((BN, C), lambda ph, i: (i * ph, 0)),
        out_shape=jax.ShapeDtypeStruct((N, C), jnp.float32),
        scratch_shapes=[pltpu.VMEM((N, C), jnp.float32),
                        pltpu.VMEM((NB, 2, C), jnp.float32)],
    )(x, w, b, g, gb, ga)


def _dense0a_wrap(x_ref, w_ref, b_ref, g_ref, gb_ref, ga_ref, h_ref,
                  t_sc, ps_sc):
    ph = pl.program_id(0)
    i = pl.program_id(1)

    @pl.when(ph == 0)
    def _():
        t = jnp.dot(x_ref[...], w_ref[...], **_DOT) + b_ref[...]
        t_sc[pl.ds(i * BN, BN), :] = t
        ps_sc[i] = _sums(t)

    @pl.when(ph == 1)
    def _():
        h_ref[...] = _norm_gelu(t_sc[pl.ds(i * BN, BN), :], ps_sc[...],
                                g_ref[...], gb_ref[...], ga_ref[...])


def _dense_mid_body(h_ref, p_ref, cnt_ref, wr_ref, wn_ref, b_ref, g_ref,
                    gb_ref, ga_ref, z_ref, t_sc, ps_sc):
    ph = pl.program_id(1)
    i = pl.program_id(2)

    @pl.when(ph == 0)
    def _():
        t = (jnp.dot(h_ref[0], wr_ref[0], **_DOT)
             + jnp.dot(_agg_block(p_ref, cnt_ref), wn_ref[0], **_DOT)
             + b_ref[0])
        t_sc[pl.ds(i * BN, BN), :] = t
        ps_sc[i] = _sums(t)

    @pl.when(ph == 1)
    def _():
        z_ref[0] = _norm_gelu(t_sc[pl.ds(i * BN, BN), :], ps_sc[...],
                              g_ref[0], gb_ref[0], ga_ref[0])


def _dense_out_body(z_ref, p_ref, cnt_ref, wr_ref, wn_ref, b_ref, o_ref):
    o_ref[0] = (jnp.dot(z_ref[0], wr_ref[0], **_DOT)
                + jnp.dot(_agg_block(p_ref, cnt_ref), wn_ref[0], **_DOT)
                + b_ref[0])


def _bspec3(per_branch, shape):
    if per_branch:
        return pl.BlockSpec((1,) + shape,
                            lambda m, ph, i: (m,) + (0,) * len(shape))
    return pl.BlockSpec((1,) + shape,
                        lambda m, ph, i: (0,) * (len(shape) + 1))


def _dense_mid(h, p, cnt, wr, wn, b, g, gb, ga, per_branch_w, per_branch_in):
    if per_branch_in:
        hspec = pl.BlockSpec((1, BN, C), lambda m, ph, i: (m, i * (1 - ph), 0))
    else:
        hspec = pl.BlockSpec((1, BN, C), lambda m, ph, i: (0, i * (1 - ph), 0))
    return pl.pallas_call(
        _dense_mid_body,
        grid=(3, 2, NB),
        in_specs=[
            hspec,
            pl.BlockSpec((NC, 1, BN, C),
                         lambda m, ph, i: (0, m, i * (1 - ph), 0)),
            pl.BlockSpec((NC, 1, BN, 1),
                         lambda m, ph, i: (0, m, i * (1 - ph), 0)),
            _bspec3(per_branch_w, (C, C)), _bspec3(per_branch_w, (C, C)),
            _bspec3(per_branch_w, (1, C)), _bspec3(per_branch_w, (1, C)),
            _bspec3(per_branch_w, (1, C)), _bspec3(per_branch_w, (1, C)),
        ],
        out_specs=pl.BlockSpec((1, BN, C), lambda m, ph, i: (m, i * ph, 0)),
        out_shape=jax.ShapeDtypeStruct((3, N, C), jnp.float32),
        scratch_shapes=[pltpu.VMEM((N, C), jnp.float32),
                        pltpu.VMEM((NB, 2, C), jnp.float32)],
    )(h, p, cnt, wr, wn, b, g, gb, ga)


def _dense_out(z, p, cnt, wr, wn, b):
    def pb(shape):
        return pl.BlockSpec((1,) + shape,
                            lambda m, i: (m,) + (0,) * len(shape))
    return pl.pallas_call(
        _dense_out_body,
        grid=(3, NB),
        in_specs=[pl.BlockSpec((1, BN, C), lambda m, i: (m, i, 0)),
                  pl.BlockSpec((NC, 1, BN, C), lambda m, i: (0, m, i, 0)),
                  pl.BlockSpec((NC, 1, BN, 1), lambda m, i: (0, m, i, 0)),
                  pb((C, FINAL)), pb((C, FINAL)), pb((1, FINAL))],
        out_specs=pl.BlockSpec((1, BN, FINAL), lambda m, i: (m, i, 0)),
        out_shape=jax.ShapeDtypeStruct((3, N, FINAL), jnp.float32),
    )(z, p, cnt, wr, wn, b)


@functools.cache
def _mesh():
    return plsc.VectorSubcoreMesh(core_axis_name="c", subcore_axis_name="s",
                                  num_cores=NC, num_subcores=NS)


def _sc_counts_body(dstp_hbm, ones_hbm, out_hbm,
                    dst_v, ones_v, zb_v, acc0, acc1, acc2, sem):
    cid = lax.axis_index("c")
    sid = lax.axis_index("s")
    w = sid * NC + cid
    accs = (acc0, acc1, acc2)

    @pl.loop(0, 128)
    def _(i):
        zb_v[i] = jnp.zeros((16,), jnp.float32)

    for acc in accs:
        for k in range(NSL // 128):
            pltpu.sync_copy(zb_v, acc.at[pl.ds(sid * NSL + k * 128, 128)])
    pltpu.sync_copy(ones_hbm, ones_v)
    for m in range(3):
        pltpu.sync_copy(dstp_hbm.at[m, pl.ds(w * CH, CH)],
                        dst_v.at[pl.ds(m * CH, CH)])
    plsc.subcore_barrier()
    for m, acc in enumerate(accs):
        @pl.loop(0, CH)
        def _(c):
            pltpu.sync_copy(ones_v, acc.at[dst_v.at[m * CH + c]], add=True)
    plsc.subcore_barrier()
    for m, acc in enumerate(accs):
        pltpu.sync_copy(acc.at[pl.ds(sid * NSL, NSL)],
                        out_hbm.at[cid, m, pl.ds(sid * NSL, NSL)])


def _sc_agg_body(stage1, table_hbm, srcp_hbm, dstp_hbm, out_hbm,
                 src_v, dst_v, rows0, rows1, zb_v, accA, sem0, sem1):
    cid = lax.axis_index("c")
    sid = lax.axis_index("s")
    w = sid * NC + cid
    for m in range(3):
        pltpu.sync_copy(dstp_hbm.at[m, pl.ds(w * CH, CH)],
                        dst_v.at[pl.ds(m * CH, CH)])
        sm = 0 if stage1 else m
        pltpu.sync_copy(srcp_hbm.at[sm, pl.ds(w * CH, CH)],
                        src_v.at[pl.ds(m * CH, CH)])

    @pl.loop(0, 128)
    def _(i):
        for j in range(0, C, 16):
            zb_v[i, pl.ds(j, 16)] = jnp.zeros((16,), jnp.float32)

    passes = ((0,), (1,), (2,))
    for masks in passes:
        accs = (accA,)[:len(masks)]
        for acc in accs:
            for k in range(NSL // 128):
                pltpu.sync_copy(zb_v, acc.at[pl.ds(sid * NSL + k * 128, 128)])
        plsc.subcore_barrier()
        base = masks[0] * CH

        def gather(c, buf, sem):
            pltpu.async_copy(table_hbm.at[src_v.at[base + c]], buf, sem)

        def gwait(buf, sem):
            pltpu.make_async_copy(table_hbm.at[pl.ds(0, 128)], buf, sem).wait()

        def scatter(c, buf):
            for k, mk in enumerate(masks):
                pltpu.sync_copy(buf, accs[k].at[dst_v.at[mk * CH + c]],
                                add=True)

        gather(0, rows0, sem0)
        gather(1, rows1, sem1)

        @pl.loop(0, CH // 2 - 1)
        def _(cc):
            c = cc * 2
            gwait(rows0, sem0)
            scatter(c, rows0)
            gather(c + 2, rows0, sem0)
            gwait(rows1, sem1)
            scatter(c + 1, rows1)
            gather(c + 3, rows1, sem1)

        gwait(rows0, sem0)
        scatter(CH - 2, rows0)
        gwait(rows1, sem1)
        scatter(CH - 1, rows1)
        plsc.subcore_barrier()
        for k, mk in enumerate(masks):
            pltpu.sync_copy(accs[k].at[pl.ds(sid * NSL, NSL)],
                            out_hbm.at[cid, mk, pl.ds(sid * NSL, NSL)])


def _make_sc_counts():
    return pl.kernel(
        _sc_counts_body,
        out_type=jax.ShapeDtypeStruct((NC, 3, NPAD, 16), jnp.float32),
        mesh=_mesh(),
        compiler_params=pltpu.CompilerParams(use_tc_tiling_on_sc=False),
        scratch_types=[
            pltpu.VMEM((3 * CH, 128), jnp.int32),
            pltpu.VMEM((128, 16), jnp.float32),
            pltpu.VMEM((128, 16), jnp.float32),
            pltpu.VMEM_SHARED((NPAD, 16), jnp.float32),
            pltpu.VMEM_SHARED((NPAD, 16), jnp.float32),
            pltpu.VMEM_SHARED((NPAD, 16), jnp.float32),
            pltpu.SemaphoreType.DMA,
        ],
    )


def _make_sc_agg(stage1):
    return pl.kernel(
        functools.partial(_sc_agg_body, stage1),
        out_type=jax.ShapeDtypeStruct((NC, 3, NPAD, C), jnp.float32),
        mesh=_mesh(),
        compiler_params=pltpu.CompilerParams(use_tc_tiling_on_sc=False),
        scratch_types=[
            pltpu.VMEM((3 * CH, 128), jnp.int32),
            pltpu.VMEM((3 * CH, 128), jnp.int32),
            pltpu.VMEM((128, C), jnp.float32),
            pltpu.VMEM((128, C), jnp.float32),
            pltpu.VMEM((128, C), jnp.float32),
            pltpu.VMEM_SHARED((NPAD, C), jnp.float32),
            pltpu.SemaphoreType.DMA,
            pltpu.SemaphoreType.DMA,
        ],
    )


def kernel(x, edge_index, edge_attr, params):
    pad = RP * 128 - E
    srcr = jnp.pad(edge_index[0], (0, pad)).reshape(RP, 128)
    dstr = jnp.pad(edge_index[1], (0, pad)).reshape(RP, 128)
    attrr = jnp.pad(edge_attr, (0, pad)).reshape(RP, 128)
    srcp, dstp = _prep(srcr, dstr, attrr)

    p011 = params["layer011"]
    gn0 = params["batch01"]
    h = _dense0(x, p011["w"], p011["b"].reshape(1, C),
                gn0["g"].reshape(1, C), gn0["b"].reshape(1, C),
                gn0["a"].reshape(1, C))

    ones16 = jnp.ones((128, 16), jnp.float32)

    cntp = _make_sc_counts()(dstp, ones16)
    cnt1 = cntp[:, :, :, 0].reshape(NC, 3, NPAD, 1)
    agg1 = _make_sc_agg(True)(h, srcp, dstp)

    def stack(names, field):
        return jnp.stack([params[n][field] for n in names])

    l1 = ("layer11", "layer12", "layer13")
    g1 = ("batch11", "batch12", "batch13")
    z1 = _dense_mid(h[None], agg1, cnt1,
                    stack(l1, "wr"), stack(l1, "wn"),
                    stack(l1, "b")[:, None, :],
                    stack(g1, "g")[:, None, :], stack(g1, "b")[:, None, :],
                    stack(g1, "a")[:, None, :],
                    per_branch_w=True, per_branch_in=False)

    agg2 = _make_sc_agg(False)(z1.reshape(3 * N, C), srcp, dstp)

    p21 = params["layer21"]
    g21 = params["batch21"]
    z2 = _dense_mid(z1, agg2, cnt1,
                    p21["wr"][None], p21["wn"][None],
                    p21["b"].reshape(1, 1, C),
                    g21["g"].reshape(1, 1, C), g21["b"].reshape(1, 1, C),
                    g21["a"].reshape(1, 1, C),
                    per_branch_w=False, per_branch_in=True)

    agg3 = _make_sc_agg(False)(z2.reshape(3 * N, C), srcp, dstp)

    l3 = ("layer31", "layer32", "layer33")
    out = _dense_out(z2, agg3, cnt1,
                     stack(l3, "wr"), stack(l3, "wn"),
                     stack(l3, "b")[:, None, :])
    return (out[0], out[1], out[2])


# speedup vs baseline: 9.5040x; 1.1603x over previous
import functools

import jax
import jax.numpy as jnp
from jax import lax
from jax.experimental import pallas as pl
from jax.experimental.pallas import tpu as pltpu
from jax.experimental.pallas import tpu_sc as plsc

N = 10000
E = 320000
D_FEAT = 128
C = 64
FINAL = 3

NC, NS, LANES = 2, 16, 16
NW = NC * NS
CH = 80
CHH = 80
RP = NW * CH
NPAD = N + 240
NSL = NPAD // NS

BN = 1000
NB = N // BN
RB = 320
PB = RP // RB

_SQRT_HALF = 0.7071067811865476
_DOT = dict(preferred_element_type=jnp.float32, precision=lax.Precision.HIGHEST)


def _gelu(z):
    return 0.5 * z * (1.0 + lax.erf(z * _SQRT_HALF))


def _prep_body(src_ref, dst_ref, attr_ref, srcp_ref, dstp_ref):
    src = src_ref[...]
    dst = dst_ref[...]
    attr = attr_ref[...]
    base = pl.program_id(0) * (RB * 128)
    epos = base + (lax.broadcasted_iota(jnp.int32, (RB, 128), 0) * 128
                   + lax.broadcasted_iota(jnp.int32, (RB, 128), 1))
    is_pad = epos >= E
    dummy = N + (epos & 127)
    s0 = jnp.where(is_pad, epos - E, src)
    srcp_ref[0] = s0
    srcp_ref[1] = s0 + N
    srcp_ref[2] = s0 + 2 * N
    dstp_ref[0] = jnp.where(jnp.logical_and(~is_pad, attr <= 0.0), dst, dummy)
    dstp_ref[1] = jnp.where(jnp.logical_and(~is_pad, attr >= 0.0), dst, dummy)
    dstp_ref[2] = jnp.where(is_pad, dummy, dst)


def _prep(srcr, dstr, attrr):
    eb = pl.BlockSpec((RB, 128), lambda i: (i, 0))
    ob = pl.BlockSpec((3, RB, 128), lambda i: (0, i, 0))
    return pl.pallas_call(
        _prep_body,
        grid=(PB,),
        in_specs=[eb, eb, eb],
        out_specs=(ob, ob),
        out_shape=(jax.ShapeDtypeStruct((3, RP, 128), jnp.int32),
                   jax.ShapeDtypeStruct((3, RP, 128), jnp.int32)),
    )(srcr, dstr, attrr)


def _sums(t):
    return jnp.concatenate([jnp.sum(t, 0, keepdims=True),
                            jnp.sum(t * t, 0, keepdims=True)], axis=0)


def _norm_gelu(t, ps, g, gb, a, eps=1e-5):
    s1 = jnp.sum(ps[:, 0, :], axis=0, keepdims=True) * (1.0 / N)
    s2 = jnp.sum(ps[:, 1, :], axis=0, keepdims=True) * (1.0 / N)
    var = s2 - a * (2.0 - a) * s1 * s1
    o = t - a * s1
    return _gelu(o * lax.rsqrt(var + eps) * g + gb)


def _agg_block(p_ref, cnt_ref):
    cnt = jnp.maximum(cnt_ref[0, 0] + cnt_ref[1, 0], 1.0)
    return (p_ref[0, 0] + p_ref[1, 0]) / cnt


def _dense0(x, w, b, g, gb, ga):
    return pl.pallas_call(
        _dense0a_wrap,
        grid=(2, NB),
        in_specs=[pl.BlockSpec((BN, D_FEAT), lambda ph, i: (i * (1 - ph), 0)),
                  pl.BlockSpec((D_FEAT, C), lambda ph, i: (0, 0)),
                  pl.BlockSpec((1, C), lambda ph, i: (0, 0)),
                  pl.BlockSpec((1, C), lambda ph, i: (0, 0)),
                  pl.BlockSpec((1, C), lambda ph, i: (0, 0)),
                  pl.BlockSpec((1, C), lambda ph, i: (0, 0))],
        out_specs=pl.BlockSpec((BN, C), lambda ph, i: (i * ph, 0)),
        out_shape=jax.ShapeDtypeStruct((N, C), jnp.float32),
        scratch_shapes=[pltpu.VMEM((N, C), jnp.float32),
                        pltpu.VMEM((NB, 2, C), jnp.float32)],
    )(x, w, b, g, gb, ga)


def _dense0a_wrap(x_ref, w_ref, b_ref, g_ref, gb_ref, ga_ref, h_ref,
                  t_sc, ps_sc):
    ph = pl.program_id(0)
    i = pl.program_id(1)

    @pl.when(ph == 0)
    def _():
        t = jnp.dot(x_ref[...], w_ref[...], **_DOT) + b_ref[...]
        t_sc[pl.ds(i * BN, BN), :] = t
        ps_sc[i] = _sums(t)

    @pl.when(ph == 1)
    def _():
        h_ref[...] = _norm_gelu(t_sc[pl.ds(i * BN, BN), :], ps_sc[...],
                                g_ref[...], gb_ref[...], ga_ref[...])


def _dense_mid_body(h_ref, p_ref, cnt_ref, wr_ref, wn_ref, b_ref, g_ref,
                    gb_ref, ga_ref, z_ref, t_sc, ps_sc):
    ph = pl.program_id(1)
    i = pl.program_id(2)

    @pl.when(ph == 0)
    def _():
        t = (jnp.dot(h_ref[0], wr_ref[0], **_DOT)
             + jnp.dot(_agg_block(p_ref, cnt_ref), wn_ref[0], **_DOT)
             + b_ref[0])
        t_sc[pl.ds(i * BN, BN), :] = t
        ps_sc[i] = _sums(t)

    @pl.when(ph == 1)
    def _():
        z_ref[0] = _norm_gelu(t_sc[pl.ds(i * BN, BN), :], ps_sc[...],
                              g_ref[0], gb_ref[0], ga_ref[0])


def _dense_out_body(z_ref, p_ref, cnt_ref, wr_ref, wn_ref, b_ref, o_ref):
    o_ref[0] = (jnp.dot(z_ref[0], wr_ref[0], **_DOT)
                + jnp.dot(_agg_block(p_ref, cnt_ref), wn_ref[0], **_DOT)
                + b_ref[0])


def _bspec3(per_branch, shape):
    if per_branch:
        return pl.BlockSpec((1,) + shape,
                            lambda m, ph, i: (m,) + (0,) * len(shape))
    return pl.BlockSpec((1,) + shape,
                        lambda m, ph, i: (0,) * (len(shape) + 1))


def _dense_mid(h, p, cnt, wr, wn, b, g, gb, ga, per_branch_w, per_branch_in):
    if per_branch_in:
        hspec = pl.BlockSpec((1, BN, C), lambda m, ph, i: (m, i * (1 - ph), 0))
    else:
        hspec = pl.BlockSpec((1, BN, C), lambda m, ph, i: (0, i * (1 - ph), 0))
    return pl.pallas_call(
        _dense_mid_body,
        grid=(3, 2, NB),
        in_specs=[
            hspec,
            pl.BlockSpec((NC, 1, BN, C),
                         lambda m, ph, i: (0, m, i * (1 - ph), 0)),
            pl.BlockSpec((NC, 1, BN, 1),
                         lambda m, ph, i: (0, m, i * (1 - ph), 0)),
            _bspec3(per_branch_w, (C, C)), _bspec3(per_branch_w, (C, C)),
            _bspec3(per_branch_w, (1, C)), _bspec3(per_branch_w, (1, C)),
            _bspec3(per_branch_w, (1, C)), _bspec3(per_branch_w, (1, C)),
        ],
        out_specs=pl.BlockSpec((1, BN, C), lambda m, ph, i: (m, i * ph, 0)),
        out_shape=jax.ShapeDtypeStruct((3, N, C), jnp.float32),
        scratch_shapes=[pltpu.VMEM((N, C), jnp.float32),
                        pltpu.VMEM((NB, 2, C), jnp.float32)],
    )(h, p, cnt, wr, wn, b, g, gb, ga)


def _dense_out(z, p, cnt, wr, wn, b):
    def pb(shape):
        return pl.BlockSpec((1,) + shape,
                            lambda m, i: (m,) + (0,) * len(shape))
    return pl.pallas_call(
        _dense_out_body,
        grid=(3, NB),
        in_specs=[pl.BlockSpec((1, BN, C), lambda m, i: (m, i, 0)),
                  pl.BlockSpec((NC, 1, BN, C), lambda m, i: (0, m, i, 0)),
                  pl.BlockSpec((NC, 1, BN, 1), lambda m, i: (0, m, i, 0)),
                  pb((C, FINAL)), pb((C, FINAL)), pb((1, FINAL))],
        out_specs=pl.BlockSpec((1, BN, FINAL), lambda m, i: (m, i, 0)),
        out_shape=jax.ShapeDtypeStruct((3, N, FINAL), jnp.float32),
    )(z, p, cnt, wr, wn, b)


@functools.cache
def _mesh():
    return plsc.VectorSubcoreMesh(core_axis_name="c", subcore_axis_name="s",
                                  num_cores=NC, num_subcores=NS)


def _sc_counts_body(dstp_hbm, ones_hbm, out_hbm,
                    dst_v, ones_v, zb_v, acc0, acc1, acc2, sem):
    cid = lax.axis_index("c")
    sid = lax.axis_index("s")
    w = sid * NC + cid
    accs = (acc0, acc1, acc2)

    @pl.loop(0, 128)
    def _(i):
        zb_v[i] = jnp.zeros((16,), jnp.float32)

    for acc in accs:
        for k in range(NSL // 128):
            pltpu.sync_copy(zb_v, acc.at[pl.ds(sid * NSL + k * 128, 128)])
    pltpu.sync_copy(ones_hbm, ones_v)
    for m in range(3):
        pltpu.sync_copy(dstp_hbm.at[m, pl.ds(w * CHH, CHH)],
                        dst_v.at[pl.ds(m * CHH, CHH)])
    plsc.subcore_barrier()
    for m, acc in enumerate(accs):
        @pl.loop(0, CHH)
        def _(c):
            pltpu.sync_copy(ones_v, acc.at[dst_v.at[m * CHH + c]], add=True)
    plsc.subcore_barrier()
    for m, acc in enumerate(accs):
        pltpu.sync_copy(acc.at[pl.ds(sid * NSL, NSL)],
                        out_hbm.at[cid, m, pl.ds(sid * NSL, NSL)])


def _make_sc_counts():
    return pl.kernel(
        _sc_counts_body,
        out_type=jax.ShapeDtypeStruct((NC, 3, NPAD, 16), jnp.float32),
        mesh=_mesh(),
        compiler_params=pltpu.CompilerParams(use_tc_tiling_on_sc=False),
        scratch_types=[
            pltpu.VMEM((3 * CHH, 128), jnp.int32),
            pltpu.VMEM((128, 16), jnp.float32),
            pltpu.VMEM((128, 16), jnp.float32),
            pltpu.VMEM_SHARED((NPAD, 16), jnp.float32),
            pltpu.VMEM_SHARED((NPAD, 16), jnp.float32),
            pltpu.VMEM_SHARED((NPAD, 16), jnp.float32),
            pltpu.SemaphoreType.DMA,
        ],
    )


def _sc_agg_body(stage1, *refs):
    (table_hbm, srcp_hbm, dstp_hbm, out_hbm, src_v, dst_v,
     rows0, rows1, rows2, rows3, zb_v, acc, sem0, sem1, sem2, sem3) = refs
    cid = lax.axis_index("c")
    sid = lax.axis_index("s")
    w = sid * NC + cid

    @pl.loop(0, 128)
    def _(i):
        for j in range(0, C, 16):
            zb_v[i, pl.ds(j, 16)] = jnp.zeros((16,), jnp.float32)

    for m in range(3):
        pltpu.sync_copy(dstp_hbm.at[m, pl.ds(w * CHH, CHH)], dst_v)
        sm = 0 if stage1 else m
        pltpu.sync_copy(srcp_hbm.at[sm, pl.ds(w * CHH, CHH)], src_v)
        for k in range(NSL // 128):
            pltpu.sync_copy(zb_v, acc.at[pl.ds(sid * NSL + k * 128, 128)])
        plsc.subcore_barrier()

        def gather(c, buf, sem):
            pltpu.async_copy(table_hbm.at[src_v.at[c]], buf, sem)

        def gwait(buf, sem):
            pltpu.make_async_copy(table_hbm.at[pl.ds(0, 128)], buf,
                                  sem).wait()

        def scatter(c, buf):
            pltpu.sync_copy(buf, acc.at[dst_v.at[c]], add=True)

        bufs = (rows0, rows1, rows2, rows3)
        sems = (sem0, sem1, sem2, sem3)
        for k in range(4):
            gather(k, bufs[k], sems[k])

        @pl.loop(0, CHH // 4 - 1)
        def _(cc):
            c = cc * 4
            for k in range(4):
                gwait(bufs[k], sems[k])
                scatter(c + k, bufs[k])
                gather(c + k + 4, bufs[k], sems[k])

        for k in range(4):
            gwait(bufs[k], sems[k])
            scatter(CHH - 4 + k, bufs[k])
        plsc.subcore_barrier()
        pltpu.sync_copy(acc.at[pl.ds(sid * NSL, NSL)],
                        out_hbm.at[cid, m, pl.ds(sid * NSL, NSL)])


def _make_sc_agg(stage1):
    return pl.kernel(
        functools.partial(_sc_agg_body, stage1),
        out_type=jax.ShapeDtypeStruct((NC, 3, NPAD, C), jnp.float32),
        mesh=_mesh(),
        compiler_params=pltpu.CompilerParams(use_tc_tiling_on_sc=False),
        scratch_types=[
            pltpu.VMEM((CHH, 128), jnp.int32),
            pltpu.VMEM((CHH, 128), jnp.int32),
            pltpu.VMEM((128, C), jnp.float32),
            pltpu.VMEM((128, C), jnp.float32),
            pltpu.VMEM((128, C), jnp.float32),
            pltpu.VMEM((128, C), jnp.float32),
            pltpu.VMEM((128, C), jnp.float32),
            pltpu.VMEM_SHARED((NPAD, C), jnp.float32),
            pltpu.SemaphoreType.DMA,
            pltpu.SemaphoreType.DMA,
            pltpu.SemaphoreType.DMA,
            pltpu.SemaphoreType.DMA,
        ],
    )


def kernel(x, edge_index, edge_attr, params):
    pad = RP * 128 - E
    srcr = jnp.pad(edge_index[0], (0, pad)).reshape(RP, 128)
    dstr = jnp.pad(edge_index[1], (0, pad)).reshape(RP, 128)
    attrr = jnp.pad(edge_attr, (0, pad)).reshape(RP, 128)
    srcp, dstp = _prep(srcr, dstr, attrr)

    p011 = params["layer011"]
    gn0 = params["batch01"]
    h = _dense0(x, p011["w"], p011["b"].reshape(1, C),
                gn0["g"].reshape(1, C), gn0["b"].reshape(1, C),
                gn0["a"].reshape(1, C))

    ones16 = jnp.ones((128, 16), jnp.float32)
    srcp2 = srcp.reshape(3, RP, 128)
    dstp2 = dstp.reshape(3, RP, 128)

    cntp = _make_sc_counts()(dstp2, ones16)
    cnt1 = cntp[:, :, :, 0].reshape(NC, 3, NPAD, 1)
    agg1 = _make_sc_agg(True)(h, srcp2, dstp2)

    def stack(names, field):
        return jnp.stack([params[n][field] for n in names])

    l1 = ("layer11", "layer12", "layer13")
    g1 = ("batch11", "batch12", "batch13")
    z1 = _dense_mid(h[None], agg1, cnt1,
                    stack(l1, "wr"), stack(l1, "wn"),
                    stack(l1, "b")[:, None, :],
                    stack(g1, "g")[:, None, :], stack(g1, "b")[:, None, :],
                    stack(g1, "a")[:, None, :],
                    per_branch_w=True, per_branch_in=False)

    agg2 = _make_sc_agg(False)(z1.reshape(3 * N, C), srcp2, dstp2)

    p21 = params["layer21"]
    g21 = params["batch21"]
    z2 = _dense_mid(z1, agg2, cnt1,
                    p21["wr"][None], p21["wn"][None],
                    p21["b"].reshape(1, 1, C),
                    g21["g"].reshape(1, 1, C), g21["b"].reshape(1, 1, C),
                    g21["a"].reshape(1, 1, C),
                    per_branch_w=False, per_branch_in=True)

    agg3 = _make_sc_agg(False)(z2.reshape(3 * N, C), srcp2, dstp2)

    l3 = ("layer31", "layer32", "layer33")
    out = _dense_out(z2, agg3, cnt1,
                     stack(l3, "wr"), stack(l3, "wn"),
                     stack(l3, "b")[:, None, :])
    return (out[0], out[1], out[2])


# speedup vs baseline: 11.8548x; 1.2474x over previous
import functools

import jax
import jax.numpy as jnp
from jax import lax
from jax.experimental import pallas as pl
from jax.experimental.pallas import tpu as pltpu
from jax.experimental.pallas import tpu_sc as plsc

N = 10000
E = 320000
D_FEAT = 128
C = 64
FINAL = 3

NC, NS, LANES = 2, 16, 16
NW = NC * NS
CHH = 80
RP = NW * CHH
NPAD = N + 240
NSL = NPAD // NS

BN = 1000
NB = N // BN
RB = 320
PB = RP // RB

_SQRT_HALF = 0.7071067811865476
_DOT = dict(preferred_element_type=jnp.float32, precision=lax.Precision.HIGHEST)


def _gelu(z):
    return 0.5 * z * (1.0 + lax.erf(z * _SQRT_HALF))


def _prep_body(src_ref, dst_ref, attr_ref, srcp_ref, dstp_ref):
    src = src_ref[...]
    dst = dst_ref[...]
    attr = attr_ref[...]
    base = pl.program_id(0) * (RB * 128)
    epos = base + (lax.broadcasted_iota(jnp.int32, (RB, 128), 0) * 128
                   + lax.broadcasted_iota(jnp.int32, (RB, 128), 1))
    is_pad = epos >= E
    dummy = N + (epos & 127)
    srcp_ref[...] = jnp.where(is_pad, epos - E, src)
    dstp_ref[0] = jnp.where(jnp.logical_and(~is_pad, attr <= 0.0), dst, dummy)
    dstp_ref[1] = jnp.where(jnp.logical_and(~is_pad, attr >= 0.0), dst, dummy)
    dstp_ref[2] = jnp.where(is_pad, dummy, dst)


def _prep(srcr, dstr, attrr):
    eb = pl.BlockSpec((RB, 128), lambda i: (i, 0))
    return pl.pallas_call(
        _prep_body,
        grid=(PB,),
        in_specs=[eb, eb, eb],
        out_specs=(eb, pl.BlockSpec((3, RB, 128), lambda i: (0, i, 0))),
        out_shape=(jax.ShapeDtypeStruct((RP, 128), jnp.int32),
                   jax.ShapeDtypeStruct((3, RP, 128), jnp.int32)),
    )(srcr, dstr, attrr)


def _sums(t):
    return jnp.concatenate([jnp.sum(t, 0, keepdims=True),
                            jnp.sum(t * t, 0, keepdims=True)], axis=0)


def _norm_gelu(t, ps, g, gb, a, eps=1e-5):
    s1 = jnp.sum(ps[:, 0, :], axis=0, keepdims=True) * (1.0 / N)
    s2 = jnp.sum(ps[:, 1, :], axis=0, keepdims=True) * (1.0 / N)
    var = s2 - a * (2.0 - a) * s1 * s1
    o = t - a * s1
    return _gelu(o * lax.rsqrt(var + eps) * g + gb)


def _agg_block(p_ref, cnt_ref):
    cnt = jnp.maximum(cnt_ref[0] + cnt_ref[1], 1.0)
    return (p_ref[0] + p_ref[1]) / cnt


def _dense0_body(x_ref, w_ref, b_ref, g_ref, gb_ref, ga_ref, h_ref,
                 t_sc, ps_sc):
    ph = pl.program_id(0)
    i = pl.program_id(1)

    @pl.when(ph == 0)
    def _():
        t = jnp.dot(x_ref[...], w_ref[...], **_DOT) + b_ref[...]
        t_sc[pl.ds(i * BN, BN), :] = t
        ps_sc[i] = _sums(t)

    @pl.when(ph == 1)
    def _():
        h_ref[...] = _norm_gelu(t_sc[pl.ds(i * BN, BN), :], ps_sc[...],
                                g_ref[...], gb_ref[...], ga_ref[...])


def _dense0(x, w, b, g, gb, ga):
    v64 = pl.BlockSpec((1, C), lambda ph, i: (0, 0))
    return pl.pallas_call(
        _dense0_body,
        grid=(2, NB),
        in_specs=[pl.BlockSpec((BN, D_FEAT), lambda ph, i: (i * (1 - ph), 0)),
                  pl.BlockSpec((D_FEAT, C), lambda ph, i: (0, 0)),
                  v64, v64, v64, v64],
        out_specs=pl.BlockSpec((BN, C), lambda ph, i: (i * ph, 0)),
        out_shape=jax.ShapeDtypeStruct((N, C), jnp.float32),
        scratch_shapes=[pltpu.VMEM((N, C), jnp.float32),
                        pltpu.VMEM((NB, 2, C), jnp.float32)],
    )(x, w, b, g, gb, ga)


def _dense_mid_body(h_ref, p_ref, cnt_ref, wr_ref, wn_ref, b_ref, g_ref,
                    gb_ref, ga_ref, z_ref, t_sc, ps_sc):
    ph = pl.program_id(0)
    i = pl.program_id(1)

    @pl.when(ph == 0)
    def _():
        t = (jnp.dot(h_ref[...], wr_ref[...], **_DOT)
             + jnp.dot(_agg_block(p_ref, cnt_ref), wn_ref[...], **_DOT)
             + b_ref[...])
        t_sc[pl.ds(i * BN, BN), :] = t
        ps_sc[i] = _sums(t)

    @pl.when(ph == 1)
    def _():
        z_ref[...] = _norm_gelu(t_sc[pl.ds(i * BN, BN), :], ps_sc[...],
                                g_ref[...], gb_ref[...], ga_ref[...])


def _dense_mid(h, p, cnt, lp, gp):
    v64 = pl.BlockSpec((1, C), lambda ph, i: (0, 0))
    nspec = pl.BlockSpec((BN, C), lambda ph, i: (i * (1 - ph), 0))
    return pl.pallas_call(
        _dense_mid_body,
        grid=(2, NB),
        in_specs=[
            nspec,
            pl.BlockSpec((NC, BN, C), lambda ph, i: (0, i * (1 - ph), 0)),
            pl.BlockSpec((NC, BN, 1), lambda ph, i: (0, i * (1 - ph), 0)),
            pl.BlockSpec((C, C), lambda ph, i: (0, 0)),
            pl.BlockSpec((C, C), lambda ph, i: (0, 0)),
            v64, v64, v64, v64,
        ],
        out_specs=pl.BlockSpec((BN, C), lambda ph, i: (i * ph, 0)),
        out_shape=jax.ShapeDtypeStruct((N, C), jnp.float32),
        scratch_shapes=[pltpu.VMEM((N, C), jnp.float32),
                        pltpu.VMEM((NB, 2, C), jnp.float32)],
    )(h, p, cnt, lp["wr"], lp["wn"], lp["b"].reshape(1, C),
      gp["g"].reshape(1, C), gp["b"].reshape(1, C), gp["a"].reshape(1, C))


def _dense_out_body(z_ref, p_ref, cnt_ref, wr_ref, wn_ref, b_ref, o_ref):
    o_ref[...] = (jnp.dot(z_ref[...], wr_ref[...], **_DOT)
                  + jnp.dot(_agg_block(p_ref, cnt_ref), wn_ref[...], **_DOT)
                  + b_ref[...])


def _dense_out(z, p, cnt, lp):
    return pl.pallas_call(
        _dense_out_body,
        grid=(NB,),
        in_specs=[pl.BlockSpec((BN, C), lambda i: (i, 0)),
                  pl.BlockSpec((NC, BN, C), lambda i: (0, i, 0)),
                  pl.BlockSpec((NC, BN, 1), lambda i: (0, i, 0)),
                  pl.BlockSpec((C, FINAL), lambda i: (0, 0)),
                  pl.BlockSpec((C, FINAL), lambda i: (0, 0)),
                  pl.BlockSpec((1, FINAL), lambda i: (0, 0))],
        out_specs=pl.BlockSpec((BN, FINAL), lambda i: (i, 0)),
        out_shape=jax.ShapeDtypeStruct((N, FINAL), jnp.float32),
    )(z, p, cnt, lp["wr"], lp["wn"], lp["b"].reshape(1, FINAL))


@functools.cache
def _mesh():
    return plsc.VectorSubcoreMesh(core_axis_name="c", subcore_axis_name="s",
                                  num_cores=NC, num_subcores=NS)


def _sc_counts_body(dstp_hbm, ones_hbm, out_hbm,
                    dst_v, ones_v, zb_v, acc0, acc1, acc2, sem):
    cid = lax.axis_index("c")
    sid = lax.axis_index("s")
    w = sid * NC + cid
    accs = (acc0, acc1, acc2)

    @pl.loop(0, 128)
    def _(i):
        zb_v[i] = jnp.zeros((16,), jnp.float32)

    for acc in accs:
        for k in range(NSL // 128):
            pltpu.sync_copy(zb_v, acc.at[pl.ds(sid * NSL + k * 128, 128)])
    pltpu.sync_copy(ones_hbm, ones_v)
    for m in range(3):
        pltpu.sync_copy(dstp_hbm.at[m, pl.ds(w * CHH, CHH)],
                        dst_v.at[pl.ds(m * CHH, CHH)])
    plsc.subcore_barrier()
    for m, acc in enumerate(accs):
        @pl.loop(0, CHH)
        def _(c):
            pltpu.sync_copy(ones_v, acc.at[dst_v.at[m * CHH + c]], add=True)
    plsc.subcore_barrier()
    for m, acc in enumerate(accs):
        pltpu.sync_copy(acc.at[pl.ds(sid * NSL, NSL)],
                        out_hbm.at[cid, m, pl.ds(sid * NSL, NSL)])


def _make_sc_counts():
    return pl.kernel(
        _sc_counts_body,
        out_type=jax.ShapeDtypeStruct((NC, 3, NPAD, 16), jnp.float32),
        mesh=_mesh(),
        compiler_params=pltpu.CompilerParams(use_tc_tiling_on_sc=False),
        scratch_types=[
            pltpu.VMEM((3 * CHH, 128), jnp.int32),
            pltpu.VMEM((128, 16), jnp.float32),
            pltpu.VMEM((128, 16), jnp.float32),
            pltpu.VMEM_SHARED((NPAD, 16), jnp.float32),
            pltpu.VMEM_SHARED((NPAD, 16), jnp.float32),
            pltpu.VMEM_SHARED((NPAD, 16), jnp.float32),
            pltpu.SemaphoreType.DMA,
        ],
    )


def _sc_agg_body(table_hbm, srcp_hbm, dstp_hbm, out_hbm, src_v, dst_v,
                 rows0, rows1, rows2, rows3, zb_v, acc, sem0, sem1, sem2,
                 sem3):
    cid = lax.axis_index("c")
    sid = lax.axis_index("s")
    w = sid * NC + cid

    @pl.loop(0, 128)
    def _(i):
        for j in range(0, C, 16):
            zb_v[i, pl.ds(j, 16)] = jnp.zeros((16,), jnp.float32)

    pltpu.sync_copy(dstp_hbm.at[pl.ds(w * CHH, CHH)], dst_v)
    pltpu.sync_copy(srcp_hbm.at[pl.ds(w * CHH, CHH)], src_v)
    for k in range(NSL // 128):
        pltpu.sync_copy(zb_v, acc.at[pl.ds(sid * NSL + k * 128, 128)])
    plsc.subcore_barrier()

    def gather(c, buf, sem):
        pltpu.async_copy(table_hbm.at[src_v.at[c]], buf, sem)

    def gwait(buf, sem):
        pltpu.make_async_copy(table_hbm.at[pl.ds(0, 128)], buf, sem).wait()

    def scatter(c, buf):
        pltpu.sync_copy(buf, acc.at[dst_v.at[c]], add=True)

    bufs = (rows0, rows1, rows2, rows3)
    sems = (sem0, sem1, sem2, sem3)
    for k in range(4):
        gather(k, bufs[k], sems[k])

    @pl.loop(0, CHH // 4 - 1)
    def _(cc):
        c = cc * 4
        for k in range(4):
            gwait(bufs[k], sems[k])
            scatter(c + k, bufs[k])
            gather(c + k + 4, bufs[k], sems[k])

    for k in range(4):
        gwait(bufs[k], sems[k])
        scatter(CHH - 4 + k, bufs[k])
    plsc.subcore_barrier()
    pltpu.sync_copy(acc.at[pl.ds(sid * NSL, NSL)],
                    out_hbm.at[cid, pl.ds(sid * NSL, NSL)])


def _make_sc_agg():
    return pl.kernel(
        _sc_agg_body,
        out_type=jax.ShapeDtypeStruct((NC, NPAD, C), jnp.float32),
        mesh=_mesh(),
        compiler_params=pltpu.CompilerParams(use_tc_tiling_on_sc=False),
        scratch_types=[
            pltpu.VMEM((CHH, 128), jnp.int32),
            pltpu.VMEM((CHH, 128), jnp.int32),
            pltpu.VMEM((128, C), jnp.float32),
            pltpu.VMEM((128, C), jnp.float32),
            pltpu.VMEM((128, C), jnp.float32),
            pltpu.VMEM((128, C), jnp.float32),
            pltpu.VMEM((128, C), jnp.float32),
            pltpu.VMEM_SHARED((NPAD, C), jnp.float32),
            pltpu.SemaphoreType.DMA,
            pltpu.SemaphoreType.DMA,
            pltpu.SemaphoreType.DMA,
            pltpu.SemaphoreType.DMA,
        ],
    )


def kernel(x, edge_index, edge_attr, params):
    pad = RP * 128 - E
    srcr = jnp.pad(edge_index[0], (0, pad)).reshape(RP, 128)
    dstr = jnp.pad(edge_index[1], (0, pad)).reshape(RP, 128)
    attrr = jnp.pad(edge_attr, (0, pad)).reshape(RP, 128)
    srcp, dstp = _prep(srcr, dstr, attrr)

    p011 = params["layer011"]
    gn0 = params["batch01"]
    h = _dense0(x, p011["w"], p011["b"].reshape(1, C),
                gn0["g"].reshape(1, C), gn0["b"].reshape(1, C),
                gn0["a"].reshape(1, C))

    ones16 = jnp.ones((128, 16), jnp.float32)
    cntp = _make_sc_counts()(dstp, ones16)
    cnts = [cntp[:, m, :, 0:1] for m in range(3)]
    dsts = [dstp[m] for m in range(3)]

    agg = _make_sc_agg()
    l1 = ("layer11", "layer12", "layer13")
    g1 = ("batch11", "batch12", "batch13")
    l3 = ("layer31", "layer32", "layer33")

    agg1 = [agg(h, srcp, dsts[m]) for m in range(3)]
    z1 = [_dense_mid(h, agg1[m], cnts[m], params[l1[m]], params[g1[m]])
          for m in range(3)]
    agg2 = [agg(z1[m], srcp, dsts[m]) for m in range(3)]
    z2 = [_dense_mid(z1[m], agg2[m], cnts[m], params["layer21"],
                     params["batch21"]) for m in range(3)]
    agg3 = [agg(z2[m], srcp, dsts[m]) for m in range(3)]
    out = [_dense_out(z2[m], agg3[m], cnts[m], params[l3[m]])
           for m in range(3)]
    return (out[0], out[1], out[2])
